# Initial kernel scaffold; baseline (speedup 1.0000x reference)
import jax
import jax.numpy as jnp
from jax.experimental import pallas as pl


def kernel(x, pos, weight_exp, bias_exp, weight_mu, bias_mu):
    raise NotImplementedError("write your pallas kernel here")


# speedup vs baseline: 2.0944x; 2.0944x over previous
import functools

import jax
import jax.numpy as jnp
from jax import lax
from jax.experimental import pallas as pl
from jax.experimental.pallas import tpu as pltpu
from jax.experimental.pallas import tpu_sc as plsc

GENE_NUM = 100000
D = 32
N = 16384
NC = 2
NS = 16
L = 16
NW = NC * NS
RPW = N // NW
CH = 128
NCH = RPW // CH


def _sc_body(we_hbm, be_hbm, wm_hbm, bm_hbm, pos_hbm, exp_hbm, flg_hbm,
             out_hbm,
             pos_v, exp_v, flg_v, idx2_v, we_v, be_v, wm_v, bm_v, out_v,
             sem0, sem1, sem2, sem3):
    wid = lax.axis_index("s") * NC + lax.axis_index("c")
    base = wid * RPW

    pltpu.sync_copy(pos_hbm.at[pl.ds(base, RPW)], pos_v)
    pltpu.sync_copy(flg_hbm.at[pl.ds(base, RPW)], flg_v)
    pltpu.sync_copy(exp_hbm.at[pl.ds(base, RPW)], exp_v)

    def mk_idx(i, carry):
        sl = pl.ds(i * L, L)
        idx2_v[sl] = pos_v[sl] * 2 + flg_v[sl].astype(jnp.int32)
        return carry
    lax.fori_loop(0, RPW // L, mk_idx, 0)

    copies = []
    for c in range(NCH):
        s = pl.ds(c * CH, CH)
        copies.append(pltpu.async_copy(we_hbm.at[pos_v.at[s]], we_v.at[s], sem0))
        copies.append(pltpu.async_copy(be_hbm.at[pos_v.at[s]], be_v.at[s], sem1))
        copies.append(pltpu.async_copy(wm_hbm.at[idx2_v.at[s]], wm_v.at[s], sem2))
        copies.append(pltpu.async_copy(bm_hbm.at[pos_v.at[s]], bm_v.at[s], sem3))
    for cp in copies:
        cp.wait()

    h0, h1 = pl.ds(0, L), pl.ds(L, L)

    def grp(g, carry):
        ev = exp_v[pl.ds(g * L, L)]
        for j in range(L):
            r = g * L + j
            e = ev[j]
            out_v[r, h0] = we_v[r, h0] * e + be_v[r, h0]
            out_v[r, h1] = we_v[r, h1] * e + be_v[r, h1]
            out_v[r, pl.ds(2 * L, L)] = wm_v[r, h0] + bm_v[r, h0]
            out_v[r, pl.ds(3 * L, L)] = wm_v[r, h1] + bm_v[r, h1]
        return carry
    lax.fori_loop(0, RPW // L, grp, 0)

    pltpu.sync_copy(out_v, out_hbm.at[pl.ds(base, RPW)])


_sc_kernel = functools.partial(
    pl.kernel,
    mesh=plsc.VectorSubcoreMesh(core_axis_name="c", subcore_axis_name="s"),
    out_type=jax.ShapeDtypeStruct((N, 2 * D), jnp.float32),
    scratch_types=[
        pltpu.VMEM((RPW,), jnp.int32),
        pltpu.VMEM((RPW,), jnp.float32),
        pltpu.VMEM((RPW,), jnp.float32),
        pltpu.VMEM((RPW,), jnp.int32),
        pltpu.VMEM((RPW, D), jnp.float32),
        pltpu.VMEM((RPW, D), jnp.float32),
        pltpu.VMEM((RPW, D), jnp.float32),
        pltpu.VMEM((RPW, D), jnp.float32),
        pltpu.VMEM((RPW, 2 * D), jnp.float32),
        pltpu.SemaphoreType.DMA,
        pltpu.SemaphoreType.DMA,
        pltpu.SemaphoreType.DMA,
        pltpu.SemaphoreType.DMA,
    ],
    compiler_params=pltpu.CompilerParams(use_tc_tiling_on_sc=False),
)(_sc_body)


def kernel(x, pos, weight_exp, bias_exp, weight_mu, bias_mu):
    pos32 = pos.astype(jnp.int32)
    exp_col = x[:, 0]
    flg_col = x[:, 1]
    wm2 = weight_mu.reshape(2 * GENE_NUM, D)
    return _sc_kernel(weight_exp, bias_exp, wm2, bias_mu, pos32,
                      exp_col, flg_col)
